# baseline (device time: 42252 ns/iter reference)
import jax
import jax.numpy as jnp
from jax import lax
from jax.experimental import pallas as pl
from jax.experimental.pallas import tpu as pltpu

N_DEV = 32


def kernel(x, Win0, Wout0, Win1, Wout1, Win2, Wout2):
    b, d_sh = x.shape
    h_dim = Win0.shape[1]
    rows = b // N_DEV
    wrows = h_dim // N_DEV

    def body(x_ref, win0, wout0, win1, wout1, win2, wout2, out_ref,
             p1_ref, wf1_src, wf2_src, l1_slots, wf1_slots, wf2_slots,
             wf1_red, wf2_red, wf1_full, wf2_full, h3_red, h3_full,
             l1rs_s, l1rs_r, w1rs_s, w1rs_r, w1ag_s, w1ag_r,
             w2rs_s, w2rs_r, w2ag_s, w2ag_r, ag3_s, ag3_r):
        my = lax.axis_index("i")

        def wf_rs_desc(src, slots, ssem, rsem, j, dst_slot):
            return pltpu.make_async_remote_copy(
                src_ref=src.at[pl.ds(j * wrows, wrows), :],
                dst_ref=slots.at[dst_slot],
                send_sem=ssem,
                recv_sem=rsem,
                device_id=(j,),
                device_id_type=pl.DeviceIdType.MESH,
            )

        def wf_ag_desc(red, full, ssem, rsem, j, dst_slot):
            return pltpu.make_async_remote_copy(
                src_ref=red,
                dst_ref=full.at[pl.ds(dst_slot * wrows, wrows), :],
                send_sem=ssem,
                recv_sem=rsem,
                device_id=(j,),
                device_id_type=pl.DeviceIdType.MESH,
            )

        def l1_rs_desc(j, dst_slot):
            return pltpu.make_async_remote_copy(
                src_ref=p1_ref.at[pl.ds(j * rows, rows), :],
                dst_ref=l1_slots.at[dst_slot],
                send_sem=l1rs_s,
                recv_sem=l1rs_r,
                device_id=(j,),
                device_id_type=pl.DeviceIdType.MESH,
            )

        def ag3_desc(j, dst_slot):
            return pltpu.make_async_remote_copy(
                src_ref=h3_red,
                dst_ref=h3_full.at[pl.ds(dst_slot * rows, rows), :],
                send_sem=ag3_s,
                recv_sem=ag3_r,
                device_id=(j,),
                device_id_type=pl.DeviceIdType.MESH,
            )

        wf1_src[...] = jnp.dot(
            wout0[...], win1[...], preferred_element_type=jnp.float32
        )
        for j in range(N_DEV):
            wf_rs_desc(wf1_src, wf1_slots, w1rs_s, w1rs_r, j, my).start()

        wf2_src[...] = jnp.dot(
            wout1[...], win2[...], preferred_element_type=jnp.float32
        )
        for j in range(N_DEV):
            wf_rs_desc(wf2_src, wf2_slots, w2rs_s, w2rs_r, j, my).start()

        p1_ref[...] = jnp.dot(
            x_ref[...], win0[...], preferred_element_type=jnp.float32
        )
        for j in range(N_DEV):
            l1_rs_desc(j, my).start()

        for i in range(N_DEV):
            wf_rs_desc(wf1_src, wf1_slots, w1rs_s, w1rs_r, i, i).wait_recv()
        wf1_red[...] = jnp.sum(wf1_slots[...], axis=0)
        for j in range(N_DEV):
            wf_ag_desc(wf1_red, wf1_full, w1ag_s, w1ag_r, j, my).start()

        for i in range(N_DEV):
            wf_rs_desc(wf2_src, wf2_slots, w2rs_s, w2rs_r, i, i).wait_recv()
        wf2_red[...] = jnp.sum(wf2_slots[...], axis=0)
        for j in range(N_DEV):
            wf_ag_desc(wf2_red, wf2_full, w2ag_s, w2ag_r, j, my).start()

        for i in range(N_DEV):
            l1_rs_desc(i, i).wait_recv()
        h1 = jnp.maximum(jnp.sum(l1_slots[...], axis=0), 0.0)

        for i in range(N_DEV):
            wf_ag_desc(wf1_red, wf1_full, w1ag_s, w1ag_r, i, i).wait_recv()
        h2 = jnp.maximum(
            jnp.dot(h1, wf1_full[...], preferred_element_type=jnp.float32), 0.0
        )
        for i in range(N_DEV):
            wf_ag_desc(wf2_red, wf2_full, w2ag_s, w2ag_r, i, i).wait_recv()
        h3_red[...] = jnp.maximum(
            jnp.dot(h2, wf2_full[...], preferred_element_type=jnp.float32), 0.0
        )

        for j in range(N_DEV):
            ag3_desc(j, my).start()
        for i in range(N_DEV):
            ag3_desc(i, i).wait_recv()
        out_ref[...] = jnp.dot(
            h3_full[...], wout2[...], preferred_element_type=jnp.float32
        )

        for j in range(N_DEV):
            wf_rs_desc(wf1_src, wf1_slots, w1rs_s, w1rs_r, j, my).wait_send()
            wf_rs_desc(wf2_src, wf2_slots, w2rs_s, w2rs_r, j, my).wait_send()
            wf_ag_desc(wf1_red, wf1_full, w1ag_s, w1ag_r, j, my).wait_send()
            wf_ag_desc(wf2_red, wf2_full, w2ag_s, w2ag_r, j, my).wait_send()
            l1_rs_desc(j, my).wait_send()
            ag3_desc(j, my).wait_send()

    return pl.pallas_call(
        body,
        out_shape=jax.ShapeDtypeStruct((b, d_sh), jnp.float32),
        in_specs=[pl.BlockSpec(memory_space=pltpu.VMEM)] * 7,
        out_specs=pl.BlockSpec(memory_space=pltpu.VMEM),
        scratch_shapes=[
            pltpu.VMEM((b, h_dim), jnp.float32),
            pltpu.VMEM((h_dim, h_dim), jnp.float32),
            pltpu.VMEM((h_dim, h_dim), jnp.float32),
            pltpu.VMEM((N_DEV, rows, h_dim), jnp.float32),
            pltpu.VMEM((N_DEV, wrows, h_dim), jnp.float32),
            pltpu.VMEM((N_DEV, wrows, h_dim), jnp.float32),
            pltpu.VMEM((wrows, h_dim), jnp.float32),
            pltpu.VMEM((wrows, h_dim), jnp.float32),
            pltpu.VMEM((h_dim, h_dim), jnp.float32),
            pltpu.VMEM((h_dim, h_dim), jnp.float32),
            pltpu.VMEM((rows, h_dim), jnp.float32),
            pltpu.VMEM((b, h_dim), jnp.float32),
            pltpu.SemaphoreType.DMA,
            pltpu.SemaphoreType.DMA,
            pltpu.SemaphoreType.DMA,
            pltpu.SemaphoreType.DMA,
            pltpu.SemaphoreType.DMA,
            pltpu.SemaphoreType.DMA,
            pltpu.SemaphoreType.DMA,
            pltpu.SemaphoreType.DMA,
            pltpu.SemaphoreType.DMA,
            pltpu.SemaphoreType.DMA,
            pltpu.SemaphoreType.DMA,
            pltpu.SemaphoreType.DMA,
        ],
    )(x, Win0, Wout0, Win1, Wout1, Win2, Wout2)


# device time: 37515 ns/iter; 1.1263x vs baseline; 1.1263x over previous
import jax
import jax.numpy as jnp
from jax import lax
from jax.experimental import pallas as pl
from jax.experimental.pallas import tpu as pltpu

N_DEV = 32


def kernel(x, Win0, Wout0, Win1, Wout1, Win2, Wout2):
    b, d_sh = x.shape
    h_dim = Win0.shape[1]
    rows = b // N_DEV
    wrows = 2 * h_dim // N_DEV

    def body(x_ref, win0, wout0, win1, wout1, win2, wout2, out_ref,
             p1_ref, wf_src, wf_slots, wf_red, wf_full,
             l1_slots, h3_red, h3_full,
             wfrs_s, wfrs_r, wfag_s, wfag_r, l1_s, l1_r, ag3_s, ag3_r):
        my = lax.axis_index("i")

        def wf_rs_desc(j, dst_slot):
            return pltpu.make_async_remote_copy(
                src_ref=wf_src.at[pl.ds(j * wrows, wrows), :],
                dst_ref=wf_slots.at[dst_slot],
                send_sem=wfrs_s,
                recv_sem=wfrs_r,
                device_id=(j,),
                device_id_type=pl.DeviceIdType.MESH,
            )

        def wf_ag_desc(j, dst_slot):
            return pltpu.make_async_remote_copy(
                src_ref=wf_red,
                dst_ref=wf_full.at[pl.ds(dst_slot * wrows, wrows), :],
                send_sem=wfag_s,
                recv_sem=wfag_r,
                device_id=(j,),
                device_id_type=pl.DeviceIdType.MESH,
            )

        def l1_rs_desc(j, dst_slot):
            return pltpu.make_async_remote_copy(
                src_ref=p1_ref.at[pl.ds(j * rows, rows), :],
                dst_ref=l1_slots.at[dst_slot],
                send_sem=l1_s,
                recv_sem=l1_r,
                device_id=(j,),
                device_id_type=pl.DeviceIdType.MESH,
            )

        def ag3_desc(j, dst_slot):
            return pltpu.make_async_remote_copy(
                src_ref=h3_red,
                dst_ref=h3_full.at[pl.ds(dst_slot * rows, rows), :],
                send_sem=ag3_s,
                recv_sem=ag3_r,
                device_id=(j,),
                device_id_type=pl.DeviceIdType.MESH,
            )

        wf_src[pl.ds(0, h_dim), :] = jnp.dot(
            wout0[...], win1[...], preferred_element_type=jnp.float32
        ).astype(jnp.bfloat16)
        wf_src[pl.ds(h_dim, h_dim), :] = jnp.dot(
            wout1[...], win2[...], preferred_element_type=jnp.float32
        ).astype(jnp.bfloat16)
        for j in range(N_DEV):
            wf_rs_desc(j, my).start()

        p1_ref[...] = jnp.dot(
            x_ref[...], win0[...], preferred_element_type=jnp.float32
        )
        for j in range(N_DEV):
            l1_rs_desc(j, my).start()

        for i in range(N_DEV):
            wf_rs_desc(i, i).wait_recv()
        wf_red[...] = jnp.sum(
            wf_slots[...].astype(jnp.float32), axis=0
        ).astype(jnp.bfloat16)
        for j in range(N_DEV):
            wf_ag_desc(j, my).start()

        for i in range(N_DEV):
            l1_rs_desc(i, i).wait_recv()
        h1 = jnp.maximum(jnp.sum(l1_slots[...], axis=0), 0.0)
        h1b = h1.astype(jnp.bfloat16)

        for i in range(N_DEV):
            wf_ag_desc(i, i).wait_recv()
        h2 = jnp.maximum(
            jnp.dot(h1b, wf_full[pl.ds(0, h_dim), :],
                    preferred_element_type=jnp.float32),
            0.0,
        ).astype(jnp.bfloat16)
        h3_red[...] = jnp.maximum(
            jnp.dot(h2, wf_full[pl.ds(h_dim, h_dim), :],
                    preferred_element_type=jnp.float32),
            0.0,
        )

        for j in range(N_DEV):
            ag3_desc(j, my).start()
        for i in range(N_DEV):
            ag3_desc(i, i).wait_recv()
        out_ref[...] = jnp.dot(
            h3_full[...], wout2[...], preferred_element_type=jnp.float32
        )

        for j in range(N_DEV):
            wf_rs_desc(j, my).wait_send()
            wf_ag_desc(j, my).wait_send()
            l1_rs_desc(j, my).wait_send()
            ag3_desc(j, my).wait_send()

    return pl.pallas_call(
        body,
        out_shape=jax.ShapeDtypeStruct((b, d_sh), jnp.float32),
        in_specs=[pl.BlockSpec(memory_space=pltpu.VMEM)] * 7,
        out_specs=pl.BlockSpec(memory_space=pltpu.VMEM),
        scratch_shapes=[
            pltpu.VMEM((b, h_dim), jnp.float32),
            pltpu.VMEM((2 * h_dim, h_dim), jnp.bfloat16),
            pltpu.VMEM((N_DEV, wrows, h_dim), jnp.bfloat16),
            pltpu.VMEM((wrows, h_dim), jnp.bfloat16),
            pltpu.VMEM((2 * h_dim, h_dim), jnp.bfloat16),
            pltpu.VMEM((N_DEV, rows, h_dim), jnp.float32),
            pltpu.VMEM((rows, h_dim), jnp.float32),
            pltpu.VMEM((b, h_dim), jnp.float32),
            pltpu.SemaphoreType.DMA,
            pltpu.SemaphoreType.DMA,
            pltpu.SemaphoreType.DMA,
            pltpu.SemaphoreType.DMA,
            pltpu.SemaphoreType.DMA,
            pltpu.SemaphoreType.DMA,
            pltpu.SemaphoreType.DMA,
            pltpu.SemaphoreType.DMA,
        ],
    )(x, Win0, Wout0, Win1, Wout1, Win2, Wout2)


# device time: 33612 ns/iter; 1.2571x vs baseline; 1.1161x over previous
import jax
import jax.numpy as jnp
from jax import lax
from jax.experimental import pallas as pl
from jax.experimental.pallas import tpu as pltpu

N_DEV = 32


def kernel(x, Win0, Wout0, Win1, Wout1, Win2, Wout2):
    b, d_sh = x.shape
    h_dim = Win0.shape[1]
    rows = b // N_DEV
    wrows = 2 * h_dim // N_DEV

    def body(x_hbm, win0_hbm, wout0_hbm, win1_hbm, wout1_hbm, win2_hbm,
             wout2_hbm, out_ref,
             x_ref, win0, wout0, win1, wout1, win2, wout2,
             p1_ref, wf_src, wf_slots, wf_red, wf_full,
             l1_slots, h3_red, h3_full,
             in_sems,
             wfrs_s, wfrs_r, wfag_s, wfag_r, l1_s, l1_r, ag3_s, ag3_r):
        my = lax.axis_index("i")

        fetches = [
            pltpu.make_async_copy(src, dst, in_sems.at[k])
            for k, (src, dst) in enumerate([
                (wout0_hbm, wout0), (win1_hbm, win1),
                (wout1_hbm, wout1), (win2_hbm, win2),
                (x_hbm, x_ref), (win0_hbm, win0),
                (wout2_hbm, wout2),
            ])
        ]
        for f in fetches:
            f.start()

        barrier = pltpu.get_barrier_semaphore()
        for j in range(N_DEV):
            pl.semaphore_signal(
                barrier, inc=1, device_id=(j,),
                device_id_type=pl.DeviceIdType.MESH,
            )
        pl.semaphore_wait(barrier, N_DEV)

        def wf_rs_desc(j, dst_slot):
            return pltpu.make_async_remote_copy(
                src_ref=wf_src.at[pl.ds(j * wrows, wrows), :],
                dst_ref=wf_slots.at[dst_slot],
                send_sem=wfrs_s,
                recv_sem=wfrs_r,
                device_id=(j,),
                device_id_type=pl.DeviceIdType.MESH,
            )

        def wf_ag_desc(j, dst_slot):
            return pltpu.make_async_remote_copy(
                src_ref=wf_red,
                dst_ref=wf_full.at[pl.ds(dst_slot * wrows, wrows), :],
                send_sem=wfag_s,
                recv_sem=wfag_r,
                device_id=(j,),
                device_id_type=pl.DeviceIdType.MESH,
            )

        def l1_rs_desc(j, dst_slot):
            return pltpu.make_async_remote_copy(
                src_ref=p1_ref.at[pl.ds(j * rows, rows), :],
                dst_ref=l1_slots.at[dst_slot],
                send_sem=l1_s,
                recv_sem=l1_r,
                device_id=(j,),
                device_id_type=pl.DeviceIdType.MESH,
            )

        def ag3_desc(j, dst_slot):
            return pltpu.make_async_remote_copy(
                src_ref=h3_red,
                dst_ref=h3_full.at[pl.ds(dst_slot * rows, rows), :],
                send_sem=ag3_s,
                recv_sem=ag3_r,
                device_id=(j,),
                device_id_type=pl.DeviceIdType.MESH,
            )

        fetches[0].wait()
        fetches[1].wait()
        wf_src[pl.ds(0, h_dim), :] = jnp.dot(
            wout0[...], win1[...], preferred_element_type=jnp.float32
        ).astype(jnp.bfloat16)
        fetches[2].wait()
        fetches[3].wait()
        wf_src[pl.ds(h_dim, h_dim), :] = jnp.dot(
            wout1[...], win2[...], preferred_element_type=jnp.float32
        ).astype(jnp.bfloat16)
        for j in range(N_DEV):
            wf_rs_desc(j, my).start()

        fetches[4].wait()
        fetches[5].wait()
        p1_ref[...] = jnp.dot(
            x_ref[...], win0[...], preferred_element_type=jnp.float32
        )
        for j in range(N_DEV):
            l1_rs_desc(j, my).start()

        for i in range(N_DEV):
            wf_rs_desc(i, i).wait_recv()
        wf_red[...] = jnp.sum(
            wf_slots[...].astype(jnp.float32), axis=0
        ).astype(jnp.bfloat16)
        for j in range(N_DEV):
            wf_ag_desc(j, my).start()

        for i in range(N_DEV):
            l1_rs_desc(i, i).wait_recv()
        h1 = jnp.maximum(jnp.sum(l1_slots[...], axis=0), 0.0)
        h1b = h1.astype(jnp.bfloat16)

        for i in range(N_DEV):
            wf_ag_desc(i, i).wait_recv()
        h2 = jnp.maximum(
            jnp.dot(h1b, wf_full[pl.ds(0, h_dim), :],
                    preferred_element_type=jnp.float32),
            0.0,
        ).astype(jnp.bfloat16)
        h3_red[...] = jnp.maximum(
            jnp.dot(h2, wf_full[pl.ds(h_dim, h_dim), :],
                    preferred_element_type=jnp.float32),
            0.0,
        )

        for j in range(N_DEV):
            ag3_desc(j, my).start()
        for i in range(N_DEV):
            ag3_desc(i, i).wait_recv()
        fetches[6].wait()
        out_ref[...] = jnp.dot(
            h3_full[...], wout2[...], preferred_element_type=jnp.float32
        )

        for j in range(N_DEV):
            wf_rs_desc(j, my).wait_send()
            wf_ag_desc(j, my).wait_send()
            l1_rs_desc(j, my).wait_send()
            ag3_desc(j, my).wait_send()

    hbm = pl.BlockSpec(memory_space=pltpu.MemorySpace.HBM)
    return pl.pallas_call(
        body,
        out_shape=jax.ShapeDtypeStruct((b, d_sh), jnp.float32),
        in_specs=[hbm] * 7,
        out_specs=pl.BlockSpec(memory_space=pltpu.VMEM),
        scratch_shapes=[
            pltpu.VMEM((b, d_sh), jnp.float32),
            pltpu.VMEM((d_sh, h_dim), jnp.float32),
            pltpu.VMEM((h_dim, d_sh), jnp.float32),
            pltpu.VMEM((d_sh, h_dim), jnp.float32),
            pltpu.VMEM((h_dim, d_sh), jnp.float32),
            pltpu.VMEM((d_sh, h_dim), jnp.float32),
            pltpu.VMEM((h_dim, d_sh), jnp.float32),
            pltpu.VMEM((b, h_dim), jnp.float32),
            pltpu.VMEM((2 * h_dim, h_dim), jnp.bfloat16),
            pltpu.VMEM((N_DEV, wrows, h_dim), jnp.bfloat16),
            pltpu.VMEM((wrows, h_dim), jnp.bfloat16),
            pltpu.VMEM((2 * h_dim, h_dim), jnp.bfloat16),
            pltpu.VMEM((N_DEV, rows, h_dim), jnp.float32),
            pltpu.VMEM((rows, h_dim), jnp.float32),
            pltpu.VMEM((b, h_dim), jnp.float32),
            pltpu.SemaphoreType.DMA((7,)),
            pltpu.SemaphoreType.DMA,
            pltpu.SemaphoreType.DMA,
            pltpu.SemaphoreType.DMA,
            pltpu.SemaphoreType.DMA,
            pltpu.SemaphoreType.DMA,
            pltpu.SemaphoreType.DMA,
            pltpu.SemaphoreType.DMA,
            pltpu.SemaphoreType.DMA,
        ],
        compiler_params=pltpu.CompilerParams(collective_id=0),
    )(x, Win0, Wout0, Win1, Wout1, Win2, Wout2)
